# Initial kernel scaffold; baseline (speedup 1.0000x reference)
import jax
import jax.numpy as jnp
from jax.experimental import pallas as pl


def kernel(node_attr, edge_index, edge_attr, edge_sh, fc_w1, fc_b1, fc_w2, fc_b2):
    raise NotImplementedError("write your pallas kernel here")


# speedup vs baseline: 1.8622x; 1.8622x over previous
import functools

import jax
import jax.numpy as jnp
import numpy as np
from jax import lax
from jax.experimental import pallas as pl
from jax.experimental.pallas import tpu as pltpu
from jax.experimental.pallas import tpu_sc as plsc

N_NODES = 10000
N_EDGES = 160000
IN_DIM = 32
OUT_DIM = 32
NEF = 16
HID = 16
ALPHA = float(1.0 / np.sqrt(IN_DIM * 1))
EPS = float(jnp.finfo(jnp.float32).eps)

NC = 2
NS = 16
NW = NC * NS
EW = N_EDGES // NW
GC = 1000
NCHUNK = EW // GC
STRIPE = N_NODES // NS

@functools.lru_cache(maxsize=None)
def _get_mesh():
    return plsc.VectorSubcoreMesh(core_axis_name="c", subcore_axis_name="s",
                                  num_cores=NC, num_subcores=NS)


def _sc_gather_body(node_hbm, dst_hbm, out_hbm, idx_v, rows_v, sem):
    wid = lax.axis_index("s") * NC + lax.axis_index("c")
    for i in range(NCHUNK):
        base = wid * EW + i * GC
        pltpu.sync_copy(dst_hbm.at[pl.ds(base, GC)], idx_v)
        pltpu.async_copy(node_hbm.at[idx_v], rows_v, sem).wait()
        pltpu.sync_copy(rows_v, out_hbm.at[pl.ds(base, GC)])


@functools.lru_cache(maxsize=None)
def _sc_gather():
    return pl.kernel(
        _sc_gather_body,
        out_type=jax.ShapeDtypeStruct((N_EDGES, IN_DIM), jnp.float32),
        mesh=_get_mesh(),
        scratch_types=[
            pltpu.VMEM((GC,), jnp.int32),
            pltpu.VMEM((GC, IN_DIM), jnp.float32),
            pltpu.SemaphoreType.DMA,
        ],
        compiler_params=pltpu.CompilerParams(use_tc_tiling_on_sc=False),
    )


def _sc_scatter_body(sum_hbm, src_hbm, z32_hbm, z16_hbm, ones_hbm,
                     psum_hbm, pcnt_hbm,
                     idx_v, val_v, ones_v, shared_sum, shared_cnt):
    cid = lax.axis_index("c")
    sid = lax.axis_index("s")
    row0 = sid * STRIPE
    pltpu.sync_copy(z32_hbm.at[pl.ds(row0, STRIPE)],
                    shared_sum.at[pl.ds(row0, STRIPE)])
    pltpu.sync_copy(z16_hbm.at[pl.ds(row0, STRIPE)],
                    shared_cnt.at[pl.ds(row0, STRIPE)])
    pltpu.sync_copy(ones_hbm, ones_v)
    plsc.subcore_barrier()
    wid = sid * NC + cid
    for i in range(NCHUNK):
        base = wid * EW + i * GC
        pltpu.sync_copy(src_hbm.at[pl.ds(base, GC)], idx_v)
        pltpu.sync_copy(sum_hbm.at[pl.ds(base, GC)], val_v)
        pltpu.sync_copy(val_v, shared_sum.at[idx_v], add=True)
        pltpu.sync_copy(ones_v, shared_cnt.at[idx_v], add=True)
    plsc.subcore_barrier()
    pltpu.sync_copy(shared_sum.at[pl.ds(row0, STRIPE)],
                    psum_hbm.at[cid, pl.ds(row0, STRIPE)])
    pltpu.sync_copy(shared_cnt.at[pl.ds(row0, STRIPE)],
                    pcnt_hbm.at[cid, pl.ds(row0, STRIPE)])


@functools.lru_cache(maxsize=None)
def _sc_scatter():
    return pl.kernel(
        _sc_scatter_body,
        out_type=(
            jax.ShapeDtypeStruct((NC, N_NODES, OUT_DIM), jnp.float32),
            jax.ShapeDtypeStruct((NC, N_NODES, HID), jnp.float32),
        ),
        mesh=_get_mesh(),
        scratch_types=[
            pltpu.VMEM((GC,), jnp.int32),
            pltpu.VMEM((GC, OUT_DIM), jnp.float32),
            pltpu.VMEM((GC, HID), jnp.float32),
            pltpu.VMEM_SHARED((N_NODES, OUT_DIM), jnp.float32),
            pltpu.VMEM_SHARED((N_NODES, HID), jnp.float32),
        ],
        compiler_params=pltpu.CompilerParams(use_tc_tiling_on_sc=False),
    )


EB = 1000


def _dense_body(ea_ref, xd_ref, sh_ref, w1t_ref, b1_ref, w2m_ref, b2r_ref,
                out_ref):
    ea = ea_ref[...]
    h = jnp.maximum(
        jnp.dot(ea, w1t_ref[...], preferred_element_type=jnp.float32)
        + b1_ref[...], 0.0)
    xd = xd_ref[...]
    g = jnp.dot(xd, w2m_ref[...], preferred_element_type=jnp.float32)
    acc = jnp.dot(xd, b2r_ref[...], preferred_element_type=jnp.float32)
    for j in range(HID):
        acc = acc + h[:, j:j + 1] * g[:, j * OUT_DIM:(j + 1) * OUT_DIM]
    out_ref[...] = (ALPHA * sh_ref[...]) * acc


def _dense(edge_attr, xd, edge_sh, w1t, b1r, w2m, b2r):
    return pl.pallas_call(
        _dense_body,
        grid=(N_EDGES // EB,),
        in_specs=[
            pl.BlockSpec((EB, NEF), lambda i: (i, 0)),
            pl.BlockSpec((EB, IN_DIM), lambda i: (i, 0)),
            pl.BlockSpec((EB, 1), lambda i: (i, 0)),
            pl.BlockSpec((NEF, HID), lambda i: (0, 0)),
            pl.BlockSpec((1, HID), lambda i: (0, 0)),
            pl.BlockSpec((IN_DIM, HID * OUT_DIM), lambda i: (0, 0)),
            pl.BlockSpec((IN_DIM, OUT_DIM), lambda i: (0, 0)),
        ],
        out_specs=pl.BlockSpec((EB, OUT_DIM), lambda i: (i, 0)),
        out_shape=jax.ShapeDtypeStruct((N_EDGES, OUT_DIM), jnp.float32),
    )(edge_attr, xd, edge_sh, w1t, b1r, w2m, b2r)


def _final_body(p_ref, c_ref, na_ref, out_ref):
    s = p_ref[0] + p_ref[1]
    cnt = c_ref[0, :, 0:1] + c_ref[1, :, 0:1]
    out_ref[...] = s / jnp.maximum(cnt, EPS) + na_ref[...]


def _final(psum, pcnt, node_attr):
    return pl.pallas_call(
        _final_body,
        out_shape=jax.ShapeDtypeStruct((N_NODES, OUT_DIM), jnp.float32),
    )(psum, pcnt, node_attr)


def kernel(node_attr, edge_index, edge_attr, edge_sh, fc_w1, fc_b1, fc_w2,
           fc_b2):
    src = edge_index[0]
    dst = edge_index[1]
    xd = _sc_gather()(node_attr, dst)
    w1t = fc_w1.T
    b1r = fc_b1.reshape(1, HID)
    w2m = fc_w2.reshape(IN_DIM, OUT_DIM, HID).transpose(0, 2, 1).reshape(
        IN_DIM, HID * OUT_DIM)
    b2r = fc_b2.reshape(IN_DIM, OUT_DIM)
    summand = _dense(edge_attr, xd, edge_sh, w1t, b1r, w2m, b2r)
    z32 = jnp.zeros((N_NODES, OUT_DIM), jnp.float32)
    z16 = jnp.zeros((N_NODES, HID), jnp.float32)
    ones = jnp.ones((GC, HID), jnp.float32)
    psum, pcnt = _sc_scatter()(summand, src, z32, z16, ones)
    return _final(psum, pcnt, node_attr)


# speedup vs baseline: 3.4603x; 1.8582x over previous
import functools

import jax
import jax.numpy as jnp
import numpy as np
from jax import lax
from jax.experimental import pallas as pl
from jax.experimental.pallas import tpu as pltpu
from jax.experimental.pallas import tpu_sc as plsc

N_NODES = 10000
N_EDGES = 160000
IN_DIM = 32
OUT_DIM = 32
NEF = 16
HID = 16
ALPHA = float(1.0 / np.sqrt(IN_DIM * 1))
EPS = float(jnp.finfo(jnp.float32).eps)

NC = 2
NS = 16
NW = NC * NS
EW = N_EDGES // NW
GC = 1000
NCHUNK = EW // GC
STRIPE = N_NODES // NS

@functools.lru_cache(maxsize=None)
def _get_mesh():
    return plsc.VectorSubcoreMesh(core_axis_name="c", subcore_axis_name="s",
                                  num_cores=NC, num_subcores=NS)


def _sc_gather_body(node_hbm, dst_hbm, out_hbm, idx_v, rows_v, sem):
    wid = lax.axis_index("s") * NC + lax.axis_index("c")
    for i in range(NCHUNK):
        base = wid * EW + i * GC
        pltpu.sync_copy(dst_hbm.at[pl.ds(base, GC)], idx_v)
        pltpu.async_copy(node_hbm.at[idx_v], rows_v, sem).wait()
        pltpu.sync_copy(rows_v, out_hbm.at[pl.ds(base, GC)])


@functools.lru_cache(maxsize=None)
def _sc_gather():
    return pl.kernel(
        _sc_gather_body,
        out_type=jax.ShapeDtypeStruct((N_EDGES, IN_DIM), jnp.float32),
        mesh=_get_mesh(),
        scratch_types=[
            pltpu.VMEM((GC,), jnp.int32),
            pltpu.VMEM((GC, IN_DIM), jnp.float32),
            pltpu.SemaphoreType.DMA,
        ],
        compiler_params=pltpu.CompilerParams(use_tc_tiling_on_sc=False),
    )


def _sc_scatter_body(sum_hbm, src_hbm, z32_hbm, z16_hbm, ones_hbm,
                     psum_hbm, pcnt_hbm,
                     idx_v, val_v, ones_v, shared_sum, shared_cnt):
    cid = lax.axis_index("c")
    sid = lax.axis_index("s")
    row0 = sid * STRIPE
    pltpu.sync_copy(z32_hbm.at[pl.ds(row0, STRIPE)],
                    shared_sum.at[pl.ds(row0, STRIPE)])
    pltpu.sync_copy(z16_hbm.at[pl.ds(row0, STRIPE)],
                    shared_cnt.at[pl.ds(row0, STRIPE)])
    pltpu.sync_copy(ones_hbm, ones_v)
    plsc.subcore_barrier()
    wid = sid * NC + cid
    for i in range(NCHUNK):
        base = wid * EW + i * GC
        pltpu.sync_copy(src_hbm.at[pl.ds(base, GC)], idx_v)
        pltpu.sync_copy(sum_hbm.at[pl.ds(base, GC)], val_v)
        pltpu.sync_copy(val_v, shared_sum.at[idx_v], add=True)
        pltpu.sync_copy(ones_v, shared_cnt.at[idx_v], add=True)
    plsc.subcore_barrier()
    pltpu.sync_copy(shared_sum.at[pl.ds(row0, STRIPE)],
                    psum_hbm.at[cid, pl.ds(row0, STRIPE)])
    pltpu.sync_copy(shared_cnt.at[pl.ds(row0, STRIPE)],
                    pcnt_hbm.at[cid, pl.ds(row0, STRIPE)])


@functools.lru_cache(maxsize=None)
def _sc_scatter():
    return pl.kernel(
        _sc_scatter_body,
        out_type=(
            jax.ShapeDtypeStruct((NC, N_NODES, OUT_DIM), jnp.float32),
            jax.ShapeDtypeStruct((NC, N_NODES, HID), jnp.float32),
        ),
        mesh=_get_mesh(),
        scratch_types=[
            pltpu.VMEM((GC,), jnp.int32),
            pltpu.VMEM((GC, OUT_DIM), jnp.float32),
            pltpu.VMEM((GC, HID), jnp.float32),
            pltpu.VMEM_SHARED((N_NODES, OUT_DIM), jnp.float32),
            pltpu.VMEM_SHARED((N_NODES, HID), jnp.float32),
        ],
        compiler_params=pltpu.CompilerParams(use_tc_tiling_on_sc=False),
    )


EB = 1000


def _dense_body(ea_ref, xd_ref, sh_ref, w1t_ref, b1_ref, w2m_ref, b2r_ref,
                r_ref, s_ref, out_ref):
    ea = ea_ref[...]
    h = jnp.maximum(
        jnp.dot(ea, w1t_ref[...], preferred_element_type=jnp.float32)
        + b1_ref[...], 0.0)
    xds = sh_ref[...] * xd_ref[...]
    g = jnp.dot(xds, w2m_ref[...], preferred_element_type=jnp.float32)
    hexp = jnp.dot(h, r_ref[...], preferred_element_type=jnp.float32)
    acc = jnp.dot(g * hexp, s_ref[...], preferred_element_type=jnp.float32)
    acc = acc + jnp.dot(xds, b2r_ref[...],
                        preferred_element_type=jnp.float32)
    out_ref[...] = acc


def _dense(edge_attr, xd, edge_sh, w1t, b1r, w2m, b2r, rmat, smat):
    return pl.pallas_call(
        _dense_body,
        grid=(N_EDGES // EB,),
        in_specs=[
            pl.BlockSpec((EB, NEF), lambda i: (i, 0)),
            pl.BlockSpec((EB, IN_DIM), lambda i: (i, 0)),
            pl.BlockSpec((EB, 1), lambda i: (i, 0)),
            pl.BlockSpec((NEF, HID), lambda i: (0, 0)),
            pl.BlockSpec((1, HID), lambda i: (0, 0)),
            pl.BlockSpec((IN_DIM, HID * OUT_DIM), lambda i: (0, 0)),
            pl.BlockSpec((IN_DIM, OUT_DIM), lambda i: (0, 0)),
            pl.BlockSpec((HID, HID * OUT_DIM), lambda i: (0, 0)),
            pl.BlockSpec((HID * OUT_DIM, OUT_DIM), lambda i: (0, 0)),
        ],
        out_specs=pl.BlockSpec((EB, OUT_DIM), lambda i: (i, 0)),
        out_shape=jax.ShapeDtypeStruct((N_EDGES, OUT_DIM), jnp.float32),
    )(edge_attr, xd, edge_sh, w1t, b1r, w2m, b2r, rmat, smat)


def _final_body(p_ref, c_ref, na_ref, out_ref):
    s = p_ref[0] + p_ref[1]
    cnt = c_ref[0, :, 0:1] + c_ref[1, :, 0:1]
    out_ref[...] = s / jnp.maximum(cnt, EPS) + na_ref[...]


def _final(psum, pcnt, node_attr):
    return pl.pallas_call(
        _final_body,
        out_shape=jax.ShapeDtypeStruct((N_NODES, OUT_DIM), jnp.float32),
    )(psum, pcnt, node_attr)


def kernel(node_attr, edge_index, edge_attr, edge_sh, fc_w1, fc_b1, fc_w2,
           fc_b2):
    src = edge_index[0]
    dst = edge_index[1]
    xd = _sc_gather()(node_attr, dst)
    w1t = fc_w1.T
    b1r = fc_b1.reshape(1, HID)
    w2m = ALPHA * fc_w2.reshape(IN_DIM, OUT_DIM, HID).transpose(0, 2, 1)\
        .reshape(IN_DIM, HID * OUT_DIM)
    b2r = ALPHA * fc_b2.reshape(IN_DIM, OUT_DIM)
    jj = np.arange(HID * OUT_DIM) // OUT_DIM
    kk = np.arange(HID * OUT_DIM) % OUT_DIM
    rmat = jnp.asarray(jj[None, :] == np.arange(HID)[:, None],
                       dtype=jnp.float32)
    smat = jnp.asarray(kk[:, None] == np.arange(OUT_DIM)[None, :],
                       dtype=jnp.float32)
    summand = _dense(edge_attr, xd, edge_sh, w1t, b1r, w2m, b2r, rmat, smat)
    z32 = jnp.zeros((N_NODES, OUT_DIM), jnp.float32)
    z16 = jnp.zeros((N_NODES, HID), jnp.float32)
    ones = jnp.ones((GC, HID), jnp.float32)
    psum, pcnt = _sc_scatter()(summand, src, z32, z16, ones)
    return _final(psum, pcnt, node_attr)


# speedup vs baseline: 4.9588x; 1.4330x over previous
import functools

import jax
import jax.numpy as jnp
import numpy as np
from jax import lax
from jax.experimental import pallas as pl
from jax.experimental.pallas import tpu as pltpu
from jax.experimental.pallas import tpu_sc as plsc

N_NODES = 10000
N_EDGES = 160000
IN_DIM = 32
OUT_DIM = 32
NEF = 16
HID = 16
ALPHA = float(1.0 / np.sqrt(IN_DIM * 1))
EPS = float(jnp.finfo(jnp.float32).eps)

NC = 2
NS = 16
NW = NC * NS
EW = N_EDGES // NW
GC = 1000
NCHUNK = EW // GC
STRIPE = N_NODES // NS

@functools.lru_cache(maxsize=None)
def _get_mesh():
    return plsc.VectorSubcoreMesh(core_axis_name="c", subcore_axis_name="s",
                                  num_cores=NC, num_subcores=NS)


def _sc_gather_body(node_hbm, dst_hbm, out_hbm, idx_v, rows_v, sem):
    wid = lax.axis_index("s") * NC + lax.axis_index("c")
    for i in range(NCHUNK):
        base = wid * EW + i * GC
        pltpu.sync_copy(dst_hbm.at[pl.ds(base, GC)], idx_v)
        pltpu.async_copy(node_hbm.at[idx_v], rows_v, sem).wait()
        pltpu.sync_copy(rows_v, out_hbm.at[pl.ds(base, GC)])


@functools.lru_cache(maxsize=None)
def _sc_gather():
    return pl.kernel(
        _sc_gather_body,
        out_type=jax.ShapeDtypeStruct((N_EDGES, IN_DIM), jnp.float32),
        mesh=_get_mesh(),
        scratch_types=[
            pltpu.VMEM((GC,), jnp.int32),
            pltpu.VMEM((GC, IN_DIM), jnp.float32),
            pltpu.SemaphoreType.DMA,
        ],
        compiler_params=pltpu.CompilerParams(use_tc_tiling_on_sc=False),
    )


def _sc_scatter_body(sum_hbm, src_hbm, z32_hbm, z16_hbm, ones_hbm,
                     psum_hbm, pcnt_hbm,
                     idx_v, val_v, ones_v, shared_sum, shared_cnt):
    cid = lax.axis_index("c")
    sid = lax.axis_index("s")
    row0 = sid * STRIPE
    pltpu.sync_copy(z32_hbm.at[pl.ds(row0, STRIPE)],
                    shared_sum.at[pl.ds(row0, STRIPE)])
    pltpu.sync_copy(z16_hbm.at[pl.ds(row0, STRIPE)],
                    shared_cnt.at[pl.ds(row0, STRIPE)])
    pltpu.sync_copy(ones_hbm, ones_v)
    plsc.subcore_barrier()
    wid = sid * NC + cid
    for i in range(NCHUNK):
        base = wid * EW + i * GC
        pltpu.sync_copy(src_hbm.at[pl.ds(base, GC)], idx_v)
        pltpu.sync_copy(sum_hbm.at[pl.ds(base, GC)], val_v)
        pltpu.sync_copy(val_v, shared_sum.at[idx_v], add=True)
        pltpu.sync_copy(ones_v, shared_cnt.at[idx_v], add=True)
    plsc.subcore_barrier()
    pltpu.sync_copy(shared_sum.at[pl.ds(row0, STRIPE)],
                    psum_hbm.at[cid, pl.ds(row0, STRIPE)])
    pltpu.sync_copy(shared_cnt.at[pl.ds(row0, STRIPE)],
                    pcnt_hbm.at[cid, pl.ds(row0, STRIPE)])


@functools.lru_cache(maxsize=None)
def _sc_scatter():
    return pl.kernel(
        _sc_scatter_body,
        out_type=(
            jax.ShapeDtypeStruct((NC, N_NODES, OUT_DIM), jnp.float32),
            jax.ShapeDtypeStruct((NC, N_NODES, HID), jnp.float32),
        ),
        mesh=_get_mesh(),
        scratch_types=[
            pltpu.VMEM((GC,), jnp.int32),
            pltpu.VMEM((GC, OUT_DIM), jnp.float32),
            pltpu.VMEM((GC, HID), jnp.float32),
            pltpu.VMEM_SHARED((N_NODES, OUT_DIM), jnp.float32),
            pltpu.VMEM_SHARED((N_NODES, HID), jnp.float32),
        ],
        compiler_params=pltpu.CompilerParams(use_tc_tiling_on_sc=False),
    )


EB = 3200


def _dense_body(eaT_ref, xdT_ref, shT_ref, w1_ref, b1c_ref, w2mT_ref,
                b2rT_ref, out_ref):
    hT = jnp.maximum(
        jnp.dot(w1_ref[...], eaT_ref[...],
                preferred_element_type=jnp.float32) + b1c_ref[...], 0.0)
    xdsT = shT_ref[...] * xdT_ref[...]
    gT = jnp.dot(w2mT_ref[...], xdsT, preferred_element_type=jnp.float32)
    acc = jnp.dot(b2rT_ref[...], xdsT, preferred_element_type=jnp.float32)
    for j in range(HID):
        acc = acc + gT[j * OUT_DIM:(j + 1) * OUT_DIM, :] * hT[j:j + 1, :]
    out_ref[...] = acc


def _dense(eaT, xdT, shT, w1, b1c, w2mT, b2rT):
    return pl.pallas_call(
        _dense_body,
        grid=(N_EDGES // EB,),
        in_specs=[
            pl.BlockSpec((NEF, EB), lambda i: (0, i)),
            pl.BlockSpec((IN_DIM, EB), lambda i: (0, i)),
            pl.BlockSpec((1, EB), lambda i: (0, i)),
            pl.BlockSpec((NEF, NEF), lambda i: (0, 0)),
            pl.BlockSpec((HID, 1), lambda i: (0, 0)),
            pl.BlockSpec((HID * OUT_DIM, IN_DIM), lambda i: (0, 0)),
            pl.BlockSpec((OUT_DIM, IN_DIM), lambda i: (0, 0)),
        ],
        out_specs=pl.BlockSpec((OUT_DIM, EB), lambda i: (0, i)),
        out_shape=jax.ShapeDtypeStruct((OUT_DIM, N_EDGES), jnp.float32),
    )(eaT, xdT, shT, w1, b1c, w2mT, b2rT)


def _final_body(p_ref, c_ref, na_ref, out_ref):
    s = p_ref[0] + p_ref[1]
    cnt = c_ref[0, :, 0:1] + c_ref[1, :, 0:1]
    out_ref[...] = s / jnp.maximum(cnt, EPS) + na_ref[...]


def _final(psum, pcnt, node_attr):
    return pl.pallas_call(
        _final_body,
        out_shape=jax.ShapeDtypeStruct((N_NODES, OUT_DIM), jnp.float32),
    )(psum, pcnt, node_attr)


def kernel(node_attr, edge_index, edge_attr, edge_sh, fc_w1, fc_b1, fc_w2,
           fc_b2):
    src = edge_index[0]
    dst = edge_index[1]
    xd = _sc_gather()(node_attr, dst)
    b1c = fc_b1.reshape(HID, 1)
    w2mT = ALPHA * fc_w2.reshape(IN_DIM, OUT_DIM, HID).transpose(2, 1, 0)\
        .reshape(HID * OUT_DIM, IN_DIM)
    b2rT = ALPHA * fc_b2.reshape(IN_DIM, OUT_DIM).T
    summandT = _dense(edge_attr.T, xd.T, edge_sh.T, fc_w1, b1c, w2mT, b2rT)
    summand = summandT.T
    z32 = jnp.zeros((N_NODES, OUT_DIM), jnp.float32)
    z16 = jnp.zeros((N_NODES, HID), jnp.float32)
    ones = jnp.ones((GC, HID), jnp.float32)
    psum, pcnt = _sc_scatter()(summand, src, z32, z16, ones)
    return _final(psum, pcnt, node_attr)
